# baseline (speedup 1.0000x reference)
import functools

import numpy as np

import jax
import jax.numpy as jnp
from jax import lax
from jax.experimental import pallas as pl
from jax.experimental.pallas import tpu as pltpu
from jax.experimental.pallas import tpu_sc as plsc

N = 10000
E = 320000
D = 128
G = 128
DH2 = 64
_SQRT_C = float(np.sqrt(np.float32(1.0 + 1e-5)))

NC, NS = 2, 16
NW = NC * NS
EPW = E // NW
CH = 80
NCH = EPW // CH
ZCH = 80
NZ = N // ZCH
NZPT = -(-NZ // NS)

R = 1000
GRID = N // R

@functools.lru_cache(maxsize=1)
def _sc_degree_kernel():
    mesh = plsc.VectorSubcoreMesh(core_axis_name="c", subcore_axis_name="s",
                                  num_cores=NC, num_subcores=NS)

    @functools.partial(
        pl.kernel,
        out_type=jax.ShapeDtypeStruct((NC, N, D), jnp.float32),
        mesh=mesh,
        scratch_types=[
            pltpu.VMEM((CH,), jnp.int32),
            pltpu.VMEM((CH, D), jnp.float32),
            pltpu.VMEM_SHARED((N, D), jnp.float32),
        ],
    )
    def deg(dst_hbm, zeros_hbm, ones_hbm, out_hbm, idx_v, ones_v, acc_sh):
        cid = lax.axis_index("c")
        sid = lax.axis_index("s")
        wid = sid * NC + cid
        pltpu.sync_copy(ones_hbm, ones_v)
        for k in range(NZPT):
            c = sid + NS * k
            @pl.when(c < NZ)
            def _():
                pltpu.sync_copy(zeros_hbm.at[pl.ds(c * ZCH, ZCH)],
                                acc_sh.at[pl.ds(c * ZCH, ZCH)])
        plsc.subcore_barrier()

        def chunk(j, carry):
            base = wid * EPW + j * CH
            pltpu.sync_copy(dst_hbm.at[pl.ds(base, CH)], idx_v)
            pltpu.sync_copy(ones_v, acc_sh.at[idx_v], add=True)
            return carry

        lax.fori_loop(0, NCH, chunk, None)
        plsc.subcore_barrier()
        for k in range(NZPT):
            c = sid + NS * k
            @pl.when(c < NZ)
            def _():
                pltpu.sync_copy(acc_sh.at[pl.ds(c * ZCH, ZCH)],
                                out_hbm.at[cid, pl.ds(c * ZCH, ZCH)])

    return deg


def _sc_degree(dst, zeros_nd, ones_ch):
    return _sc_degree_kernel()(dst, zeros_nd, ones_ch)


@functools.lru_cache(maxsize=1)
def _sc_scatter_kernel():
    mesh = plsc.VectorSubcoreMesh(core_axis_name="c", subcore_axis_name="s",
                                  num_cores=NC, num_subcores=NS)

    @functools.partial(
        pl.kernel,
        out_type=jax.ShapeDtypeStruct((NC, N, D), jnp.float32),
        mesh=mesh,
        scratch_types=[
            pltpu.VMEM((EPW,), jnp.int32),
            pltpu.VMEM((CH,), jnp.int32),
            pltpu.VMEM((CH,), jnp.int32),
            pltpu.VMEM((CH, D), jnp.float32),
            pltpu.VMEM((CH, D), jnp.float32),
            pltpu.VMEM_SHARED((N, D), jnp.float32),
            pltpu.SemaphoreType.DMA,
            pltpu.SemaphoreType.DMA,
        ],
    )
    def scat(u_hbm, src_hbm, dst_hbm, zeros_hbm, out_hbm,
             srcall_v, dst_v0, dst_v1, rows_v0, rows_v1, acc_sh,
             sem0, sem1):
        cid = lax.axis_index("c")
        sid = lax.axis_index("s")
        wid = sid * NC + cid
        ebase = wid * EPW
        pltpu.sync_copy(src_hbm.at[pl.ds(ebase, EPW)], srcall_v)
        for k in range(NZPT):
            c = sid + NS * k
            @pl.when(c < NZ)
            def _():
                pltpu.sync_copy(zeros_hbm.at[pl.ds(c * ZCH, ZCH)],
                                acc_sh.at[pl.ds(c * ZCH, ZCH)])
        plsc.subcore_barrier()

        def gather_start(j, rows_v, sem):
            pltpu.async_copy(u_hbm.at[srcall_v.at[pl.ds(j * CH, CH)]],
                             rows_v, sem)

        def gather_wait(j, rows_v, sem):
            pltpu.make_async_copy(u_hbm.at[srcall_v.at[pl.ds(j * CH, CH)]],
                                  rows_v, sem).wait()

        pltpu.sync_copy(dst_hbm.at[pl.ds(ebase, CH)], dst_v0)
        gather_start(0, rows_v0, sem0)

        def pair(t, carry):
            j1 = 2 * t + 1
            j2 = 2 * t + 2
            pltpu.sync_copy(dst_hbm.at[pl.ds(ebase + j1 * CH, CH)], dst_v1)
            gather_start(j1, rows_v1, sem1)
            gather_wait(j1 - 1, rows_v0, sem0)
            pltpu.sync_copy(rows_v0, acc_sh.at[dst_v0], add=True)
            pltpu.sync_copy(dst_hbm.at[pl.ds(ebase + j2 * CH, CH)], dst_v0)
            gather_start(j2, rows_v0, sem0)
            gather_wait(j1, rows_v1, sem1)
            pltpu.sync_copy(rows_v1, acc_sh.at[dst_v1], add=True)
            return carry

        lax.fori_loop(0, (NCH - 1) // 2, pair, None)
        gather_wait(NCH - 1, rows_v0, sem0)
        pltpu.sync_copy(rows_v0, acc_sh.at[dst_v0], add=True)
        plsc.subcore_barrier()
        for k in range(NZPT):
            c = sid + NS * k
            @pl.when(c < NZ)
            def _():
                pltpu.sync_copy(acc_sh.at[pl.ds(c * ZCH, ZCH)],
                                out_hbm.at[cid, pl.ds(c * ZCH, ZCH)])

    return scat


def _sc_scatter(u, src, dst, zeros_nd):
    return _sc_scatter_kernel()(u, src, dst, zeros_nd)


def _k1_body(x_ref, w_ref, d0_ref, d1_ref, u_ref, dinv_ref):
    deg = 1.0 + d0_ref[...][:, :1] + d1_ref[...][:, :1]
    dinv = lax.rsqrt(deg)
    t = jnp.dot(x_ref[...], w_ref[...], preferred_element_type=jnp.float32)
    u_ref[...] = t * dinv
    dinv_ref[...] = jnp.broadcast_to(dinv, (R, D))


def _k1(x, W1, d0, d1):
    return pl.pallas_call(
        _k1_body,
        grid=(GRID,),
        in_specs=[
            pl.BlockSpec((R, D), lambda i: (i, 0)),
            pl.BlockSpec((D, D), lambda i: (0, 0)),
            pl.BlockSpec((R, D), lambda i: (i, 0)),
            pl.BlockSpec((R, D), lambda i: (i, 0)),
        ],
        out_specs=[pl.BlockSpec((R, D), lambda i: (i, 0)),
                   pl.BlockSpec((R, D), lambda i: (i, 0))],
        out_shape=[jax.ShapeDtypeStruct((N, D), jnp.float32),
                   jax.ShapeDtypeStruct((N, D), jnp.float32)],
    )(x, W1, d0, d1)


def _k23_body(p_ref, u_ref, dinv_ref, b_ref, g_ref, be_ref, w_ref, uo_ref):
    p = p_ref[...]
    s = p[0] + p[1] + u_ref[...]
    dinv = dinv_ref[...]
    conv = dinv * s + b_ref[...]
    h = jnp.maximum(g_ref[...] * conv / _SQRT_C + be_ref[...], 0.0)
    uo_ref[...] = dinv * jnp.dot(h, w_ref[...],
                                 preferred_element_type=jnp.float32)


def _k23(p, u, dinv_b, b, g, be, W):
    return pl.pallas_call(
        _k23_body,
        grid=(GRID,),
        in_specs=[
            pl.BlockSpec((NC, R, D), lambda i: (0, i, 0)),
            pl.BlockSpec((R, D), lambda i: (i, 0)),
            pl.BlockSpec((R, D), lambda i: (i, 0)),
            pl.BlockSpec((1, D), lambda i: (0, 0)),
            pl.BlockSpec((1, D), lambda i: (0, 0)),
            pl.BlockSpec((1, D), lambda i: (0, 0)),
            pl.BlockSpec((D, D), lambda i: (0, 0)),
        ],
        out_specs=pl.BlockSpec((R, D), lambda i: (i, 0)),
        out_shape=jax.ShapeDtypeStruct((N, D), jnp.float32),
    )(p, u, dinv_b, b, g, be, W)


def _k4_body(batch3_ref, p_ref, u_ref, dinv_ref, b_ref, g_ref, be_ref,
             wm1_ref, bm1_ref, gm_ref, bem_ref, wm2_ref, bm2_ref,
             out_ref, acc_s, acc_c):
    i = pl.program_id(0)

    @pl.when(i == 0)
    def _():
        acc_s[...] = jnp.zeros_like(acc_s)
        acc_c[...] = jnp.zeros_like(acc_c)

    p = p_ref[...]
    s = p[0] + p[1] + u_ref[...]
    dinv = dinv_ref[...]
    conv = dinv * s + b_ref[...]
    h = jnp.maximum(g_ref[...] * conv / _SQRT_C + be_ref[...], 0.0)
    gids = lax.broadcasted_iota(jnp.int32, (G, R), 0)
    m = (gids == jnp.broadcast_to(batch3_ref[...][0], (G, R))).astype(jnp.float32)
    acc_s[...] += jnp.dot(m, h, preferred_element_type=jnp.float32, precision=lax.Precision.HIGHEST)
    acc_c[...] += jnp.broadcast_to(jnp.sum(m, axis=1, keepdims=True), (G, D))

    @pl.when(i == GRID - 1)
    def _():
        pooled = acc_s[...] / jnp.maximum(acc_c[...], 1.0)
        mlin = jnp.dot(pooled, wm1_ref[...],
                       preferred_element_type=jnp.float32) + bm1_ref[...]
        mm = jnp.maximum(gm_ref[...] * mlin / _SQRT_C + bem_ref[...], 0.0)
        out_ref[...] = (jnp.dot(mm, wm2_ref[...],
                                preferred_element_type=jnp.float32)
                        + bm2_ref[...][:, :1])


def _k4(batch3d, p, u, dinv_b, b, g, be, Wm1, bm1, gm, bem, wm2row, bm2b):
    return pl.pallas_call(
        _k4_body,
        grid=(GRID,),
        in_specs=[
            pl.BlockSpec((1, 1, R), lambda i: (i, 0, 0)),
            pl.BlockSpec((NC, R, D), lambda i: (0, i, 0)),
            pl.BlockSpec((R, D), lambda i: (i, 0)),
            pl.BlockSpec((R, D), lambda i: (i, 0)),
            pl.BlockSpec((1, D), lambda i: (0, 0)),
            pl.BlockSpec((1, D), lambda i: (0, 0)),
            pl.BlockSpec((1, D), lambda i: (0, 0)),
            pl.BlockSpec((D, DH2), lambda i: (0, 0)),
            pl.BlockSpec((1, DH2), lambda i: (0, 0)),
            pl.BlockSpec((1, DH2), lambda i: (0, 0)),
            pl.BlockSpec((1, DH2), lambda i: (0, 0)),
            pl.BlockSpec((DH2, 1), lambda i: (0, 0)),
            pl.BlockSpec((1, D), lambda i: (0, 0)),
        ],
        out_specs=pl.BlockSpec((G, 1), lambda i: (0, 0)),
        out_shape=jax.ShapeDtypeStruct((G, 1), jnp.float32),
        scratch_shapes=[
            pltpu.VMEM((G, D), jnp.float32),
            pltpu.VMEM((G, D), jnp.float32),
        ],
    )(batch3d, p, u, dinv_b, b, g, be, Wm1, bm1, gm, bem, wm2row, bm2b)


def kernel(x, edge_index, batch, W1, b1, g1, be1, W2, b2, g2, be2,
           W3, b3, g3, be3, Wm1, bm1, gm, bem, Wm2, bm2):
    src = edge_index[0]
    dst = edge_index[1]
    zeros_nd = jnp.zeros((N, D), jnp.float32)
    ones_ch = jnp.ones((CH, D), jnp.float32)

    degp = _sc_degree(dst, zeros_nd, ones_ch)
    d0 = degp[0]
    d1 = degp[1]

    u1, dinv_b = _k1(x, W1, d0, d1)
    p1 = _sc_scatter(u1, src, dst, zeros_nd)
    u2 = _k23(p1, u1, dinv_b, b1.reshape(1, D), g1.reshape(1, D),
              be1.reshape(1, D), W2)
    p2 = _sc_scatter(u2, src, dst, zeros_nd)
    u3 = _k23(p2, u2, dinv_b, b2.reshape(1, D), g2.reshape(1, D),
              be2.reshape(1, D), W3)
    p3 = _sc_scatter(u3, src, dst, zeros_nd)

    out = _k4(batch.reshape(GRID, 1, R), p3, u3, dinv_b,
              b3.reshape(1, D), g3.reshape(1, D), be3.reshape(1, D),
              Wm1, bm1.reshape(1, DH2), gm.reshape(1, DH2),
              bem.reshape(1, DH2), Wm2,
              jnp.broadcast_to(bm2.reshape(1, 1), (1, D)))
    return out


# speedup vs baseline: 1.2535x; 1.2535x over previous
import functools

import numpy as np

import jax
import jax.numpy as jnp
from jax import lax
from jax.experimental import pallas as pl
from jax.experimental.pallas import tpu as pltpu
from jax.experimental.pallas import tpu_sc as plsc

N = 10000
E = 320000
D = 128
G = 128
DH2 = 64
_SQRT_C = float(np.sqrt(np.float32(1.0 + 1e-5)))

NC, NS = 2, 16
NW = NC * NS
EPW = E // NW
CH = 80
NCH = EPW // CH
ZCH = 80
NZ = N // ZCH
NZPT = -(-NZ // NS)

R = 1000
GRID = N // R

@functools.lru_cache(maxsize=1)
def _sc_degree_kernel():
    mesh = plsc.VectorSubcoreMesh(core_axis_name="c", subcore_axis_name="s",
                                  num_cores=NC, num_subcores=NS)

    @functools.partial(
        pl.kernel,
        out_type=jax.ShapeDtypeStruct((NC, N, D), jnp.float32),
        mesh=mesh,
        scratch_types=[
            pltpu.VMEM((CH,), jnp.int32),
            pltpu.VMEM((CH,), jnp.int32),
            pltpu.VMEM((CH, D), jnp.float32),
            pltpu.VMEM_SHARED((N, D), jnp.float32),
            pltpu.SemaphoreType.DMA,
            pltpu.SemaphoreType.DMA,
        ],
    )
    def deg(dst_hbm, zeros_hbm, ones_hbm, out_hbm, idx_v0, idx_v1, ones_v,
            acc_sh, semd0, semd1):
        cid = lax.axis_index("c")
        sid = lax.axis_index("s")
        wid = sid * NC + cid
        ebase = wid * EPW
        pltpu.sync_copy(ones_hbm, ones_v)

        def idx_start(j, buf, sem):
            pltpu.async_copy(dst_hbm.at[pl.ds(ebase + j * CH, CH)], buf, sem)

        def idx_wait(j, buf, sem):
            pltpu.make_async_copy(dst_hbm.at[pl.ds(ebase + j * CH, CH)],
                                  buf, sem).wait()

        idx_start(0, idx_v0, semd0)
        for k in range(NZPT):
            c = sid + NS * k
            @pl.when(c < NZ)
            def _():
                pltpu.sync_copy(zeros_hbm.at[pl.ds(c * ZCH, ZCH)],
                                acc_sh.at[pl.ds(c * ZCH, ZCH)])
        plsc.subcore_barrier()

        def pair(t, carry):
            j1 = 2 * t + 1
            j2 = 2 * t + 2
            idx_start(j1, idx_v1, semd1)
            idx_wait(j1 - 1, idx_v0, semd0)
            pltpu.sync_copy(ones_v, acc_sh.at[idx_v0], add=True)
            idx_start(j2, idx_v0, semd0)
            idx_wait(j1, idx_v1, semd1)
            pltpu.sync_copy(ones_v, acc_sh.at[idx_v1], add=True)
            return carry

        lax.fori_loop(0, (NCH - 1) // 2, pair, None)
        idx_wait(NCH - 1, idx_v0, semd0)
        pltpu.sync_copy(ones_v, acc_sh.at[idx_v0], add=True)
        plsc.subcore_barrier()
        for k in range(NZPT):
            c = sid + NS * k
            @pl.when(c < NZ)
            def _():
                pltpu.sync_copy(acc_sh.at[pl.ds(c * ZCH, ZCH)],
                                out_hbm.at[cid, pl.ds(c * ZCH, ZCH)])

    return deg


def _sc_degree(dst, zeros_nd, ones_ch):
    return _sc_degree_kernel()(dst, zeros_nd, ones_ch)


@functools.lru_cache(maxsize=1)
def _sc_scatter_kernel():
    mesh = plsc.VectorSubcoreMesh(core_axis_name="c", subcore_axis_name="s",
                                  num_cores=NC, num_subcores=NS)

    @functools.partial(
        pl.kernel,
        out_type=jax.ShapeDtypeStruct((NC, N, D), jnp.float32),
        mesh=mesh,
        scratch_types=[
            pltpu.VMEM((EPW,), jnp.int32),
            pltpu.VMEM((CH,), jnp.int32),
            pltpu.VMEM((CH,), jnp.int32),
            pltpu.VMEM((CH, D), jnp.float32),
            pltpu.VMEM((CH, D), jnp.float32),
            pltpu.VMEM_SHARED((N, D), jnp.float32),
            pltpu.SemaphoreType.DMA,
            pltpu.SemaphoreType.DMA,
            pltpu.SemaphoreType.DMA,
            pltpu.SemaphoreType.DMA,
        ],
    )
    def scat(u_hbm, src_hbm, dst_hbm, zeros_hbm, out_hbm,
             srcall_v, dst_v0, dst_v1, rows_v0, rows_v1, acc_sh,
             sem0, sem1, semd0, semd1):
        cid = lax.axis_index("c")
        sid = lax.axis_index("s")
        wid = sid * NC + cid
        ebase = wid * EPW
        pltpu.sync_copy(src_hbm.at[pl.ds(ebase, EPW)], srcall_v)

        def gather_start(j, rows_v, sem):
            pltpu.async_copy(u_hbm.at[srcall_v.at[pl.ds(j * CH, CH)]],
                             rows_v, sem)

        def gather_wait(j, rows_v, sem):
            pltpu.make_async_copy(u_hbm.at[srcall_v.at[pl.ds(j * CH, CH)]],
                                  rows_v, sem).wait()

        def idx_start(j, buf, sem):
            pltpu.async_copy(dst_hbm.at[pl.ds(ebase + j * CH, CH)], buf, sem)

        def idx_wait(j, buf, sem):
            pltpu.make_async_copy(dst_hbm.at[pl.ds(ebase + j * CH, CH)],
                                  buf, sem).wait()

        idx_start(0, dst_v0, semd0)
        gather_start(0, rows_v0, sem0)
        for k in range(NZPT):
            c = sid + NS * k
            @pl.when(c < NZ)
            def _():
                pltpu.sync_copy(zeros_hbm.at[pl.ds(c * ZCH, ZCH)],
                                acc_sh.at[pl.ds(c * ZCH, ZCH)])
        plsc.subcore_barrier()

        def pair(t, carry):
            j1 = 2 * t + 1
            j2 = 2 * t + 2
            idx_start(j1, dst_v1, semd1)
            gather_start(j1, rows_v1, sem1)
            idx_wait(j1 - 1, dst_v0, semd0)
            gather_wait(j1 - 1, rows_v0, sem0)
            pltpu.sync_copy(rows_v0, acc_sh.at[dst_v0], add=True)
            idx_start(j2, dst_v0, semd0)
            gather_start(j2, rows_v0, sem0)
            idx_wait(j1, dst_v1, semd1)
            gather_wait(j1, rows_v1, sem1)
            pltpu.sync_copy(rows_v1, acc_sh.at[dst_v1], add=True)
            return carry

        lax.fori_loop(0, (NCH - 1) // 2, pair, None)
        idx_wait(NCH - 1, dst_v0, semd0)
        gather_wait(NCH - 1, rows_v0, sem0)
        pltpu.sync_copy(rows_v0, acc_sh.at[dst_v0], add=True)
        plsc.subcore_barrier()
        for k in range(NZPT):
            c = sid + NS * k
            @pl.when(c < NZ)
            def _():
                pltpu.sync_copy(acc_sh.at[pl.ds(c * ZCH, ZCH)],
                                out_hbm.at[cid, pl.ds(c * ZCH, ZCH)])

    return scat


def _sc_scatter(u, src, dst, zeros_nd):
    return _sc_scatter_kernel()(u, src, dst, zeros_nd)


def _k1_body(x_ref, w_ref, d0_ref, d1_ref, u_ref, dinv_ref):
    deg = 1.0 + d0_ref[...][:, :1] + d1_ref[...][:, :1]
    dinv = lax.rsqrt(deg)
    t = jnp.dot(x_ref[...], w_ref[...], preferred_element_type=jnp.float32)
    u_ref[...] = t * dinv
    dinv_ref[...] = jnp.broadcast_to(dinv, (R, D))


def _k1(x, W1, d0, d1):
    return pl.pallas_call(
        _k1_body,
        grid=(GRID,),
        in_specs=[
            pl.BlockSpec((R, D), lambda i: (i, 0)),
            pl.BlockSpec((D, D), lambda i: (0, 0)),
            pl.BlockSpec((R, D), lambda i: (i, 0)),
            pl.BlockSpec((R, D), lambda i: (i, 0)),
        ],
        out_specs=[pl.BlockSpec((R, D), lambda i: (i, 0)),
                   pl.BlockSpec((R, D), lambda i: (i, 0))],
        out_shape=[jax.ShapeDtypeStruct((N, D), jnp.float32),
                   jax.ShapeDtypeStruct((N, D), jnp.float32)],
    )(x, W1, d0, d1)


def _k23_body(p_ref, u_ref, dinv_ref, b_ref, g_ref, be_ref, w_ref, uo_ref):
    p = p_ref[...]
    s = p[0] + p[1] + u_ref[...]
    dinv = dinv_ref[...]
    conv = dinv * s + b_ref[...]
    h = jnp.maximum(g_ref[...] * conv / _SQRT_C + be_ref[...], 0.0)
    uo_ref[...] = dinv * jnp.dot(h, w_ref[...],
                                 preferred_element_type=jnp.float32)


def _k23(p, u, dinv_b, b, g, be, W):
    return pl.pallas_call(
        _k23_body,
        grid=(GRID,),
        in_specs=[
            pl.BlockSpec((NC, R, D), lambda i: (0, i, 0)),
            pl.BlockSpec((R, D), lambda i: (i, 0)),
            pl.BlockSpec((R, D), lambda i: (i, 0)),
            pl.BlockSpec((1, D), lambda i: (0, 0)),
            pl.BlockSpec((1, D), lambda i: (0, 0)),
            pl.BlockSpec((1, D), lambda i: (0, 0)),
            pl.BlockSpec((D, D), lambda i: (0, 0)),
        ],
        out_specs=pl.BlockSpec((R, D), lambda i: (i, 0)),
        out_shape=jax.ShapeDtypeStruct((N, D), jnp.float32),
    )(p, u, dinv_b, b, g, be, W)


def _k4_body(batch3_ref, p_ref, u_ref, dinv_ref, b_ref, g_ref, be_ref,
             wm1_ref, bm1_ref, gm_ref, bem_ref, wm2_ref, bm2_ref,
             out_ref, acc_s, acc_c):
    i = pl.program_id(0)

    @pl.when(i == 0)
    def _():
        acc_s[...] = jnp.zeros_like(acc_s)
        acc_c[...] = jnp.zeros_like(acc_c)

    p = p_ref[...]
    s = p[0] + p[1] + u_ref[...]
    dinv = dinv_ref[...]
    conv = dinv * s + b_ref[...]
    h = jnp.maximum(g_ref[...] * conv / _SQRT_C + be_ref[...], 0.0)
    gids = lax.broadcasted_iota(jnp.int32, (G, R), 0)
    m = (gids == jnp.broadcast_to(batch3_ref[...][0], (G, R))).astype(jnp.float32)
    acc_s[...] += jnp.dot(m, h, preferred_element_type=jnp.float32, precision=lax.Precision.HIGHEST)
    acc_c[...] += jnp.broadcast_to(jnp.sum(m, axis=1, keepdims=True), (G, D))

    @pl.when(i == GRID - 1)
    def _():
        pooled = acc_s[...] / jnp.maximum(acc_c[...], 1.0)
        mlin = jnp.dot(pooled, wm1_ref[...],
                       preferred_element_type=jnp.float32) + bm1_ref[...]
        mm = jnp.maximum(gm_ref[...] * mlin / _SQRT_C + bem_ref[...], 0.0)
        out_ref[...] = (jnp.dot(mm, wm2_ref[...],
                                preferred_element_type=jnp.float32)
                        + bm2_ref[...][:, :1])


def _k4(batch3d, p, u, dinv_b, b, g, be, Wm1, bm1, gm, bem, wm2row, bm2b):
    return pl.pallas_call(
        _k4_body,
        grid=(GRID,),
        in_specs=[
            pl.BlockSpec((1, 1, R), lambda i: (i, 0, 0)),
            pl.BlockSpec((NC, R, D), lambda i: (0, i, 0)),
            pl.BlockSpec((R, D), lambda i: (i, 0)),
            pl.BlockSpec((R, D), lambda i: (i, 0)),
            pl.BlockSpec((1, D), lambda i: (0, 0)),
            pl.BlockSpec((1, D), lambda i: (0, 0)),
            pl.BlockSpec((1, D), lambda i: (0, 0)),
            pl.BlockSpec((D, DH2), lambda i: (0, 0)),
            pl.BlockSpec((1, DH2), lambda i: (0, 0)),
            pl.BlockSpec((1, DH2), lambda i: (0, 0)),
            pl.BlockSpec((1, DH2), lambda i: (0, 0)),
            pl.BlockSpec((DH2, 1), lambda i: (0, 0)),
            pl.BlockSpec((1, D), lambda i: (0, 0)),
        ],
        out_specs=pl.BlockSpec((G, 1), lambda i: (0, 0)),
        out_shape=jax.ShapeDtypeStruct((G, 1), jnp.float32),
        scratch_shapes=[
            pltpu.VMEM((G, D), jnp.float32),
            pltpu.VMEM((G, D), jnp.float32),
        ],
    )(batch3d, p, u, dinv_b, b, g, be, Wm1, bm1, gm, bem, wm2row, bm2b)


def kernel(x, edge_index, batch, W1, b1, g1, be1, W2, b2, g2, be2,
           W3, b3, g3, be3, Wm1, bm1, gm, bem, Wm2, bm2):
    src = edge_index[0]
    dst = edge_index[1]
    zeros_nd = jnp.zeros((N, D), jnp.float32)
    ones_ch = jnp.ones((CH, D), jnp.float32)

    degp = _sc_degree(dst, zeros_nd, ones_ch)
    d0 = degp[0]
    d1 = degp[1]

    u1, dinv_b = _k1(x, W1, d0, d1)
    p1 = _sc_scatter(u1, src, dst, zeros_nd)
    u2 = _k23(p1, u1, dinv_b, b1.reshape(1, D), g1.reshape(1, D),
              be1.reshape(1, D), W2)
    p2 = _sc_scatter(u2, src, dst, zeros_nd)
    u3 = _k23(p2, u2, dinv_b, b2.reshape(1, D), g2.reshape(1, D),
              be2.reshape(1, D), W3)
    p3 = _sc_scatter(u3, src, dst, zeros_nd)

    out = _k4(batch.reshape(GRID, 1, R), p3, u3, dinv_b,
              b3.reshape(1, D), g3.reshape(1, D), be3.reshape(1, D),
              Wm1, bm1.reshape(1, DH2), gm.reshape(1, DH2),
              bem.reshape(1, DH2), Wm2,
              jnp.broadcast_to(bm2.reshape(1, 1), (1, D)))
    return out
